# baseline (speedup 1.0000x reference)
import functools

import jax
import jax.numpy as jnp
from jax import lax
from jax.experimental import pallas as pl
from jax.experimental.pallas import tpu as pltpu
from jax.experimental.pallas import tpu_sc as plsc

_PS = 16
_C = 3
_H = 512
_W = 512
_N = 65536
_WP = _W - _PS + 1
_NROWS = _N * _C * _PS

_NT = 32
_BH = _H // _NT
_BAND = _C * _BH * _W
_TRASH = _BAND

_OCS = 8192
_NOC = _N // _OCS
_RF = 2048
_RCAP = _RF + _C * _PS * 16
_SEG = 256


def _off_body(inds_ref, off_ref):
    i = inds_ref[...]
    y0 = i // _WP
    off_ref[...] = y0 * _W + (i - y0 * _WP)


def _compute_offsets(inds):
    inds2 = inds.reshape(_N // 128, 128)
    out = pl.pallas_call(
        _off_body,
        out_shape=jax.ShapeDtypeStruct((_N // 128, 128), jnp.int32),
    )(inds2)
    return out.reshape(_N)


def _sc_body(pf_hbm, off_hbm, out_hbm,
             offs_v, klo, klp, rid, rdst, gbuf, canvas, sem):
    cid = lax.axis_index("c")
    sid = lax.axis_index("s")
    w = cid * (_NT // 2) + sid
    iota = lax.broadcasted_iota(jnp.int32, (16,), 0)
    zeros16 = jnp.zeros((16,), jnp.float32)

    def _z(i, c):
        canvas[pl.ds(i * 16, 16)] = zeros16
        return c

    lax.fori_loop(0, (_BAND + 16) // 16, _z, 0)

    def _zr(i, c):
        rid[pl.ds(i * 16, 16)] = iota * 0
        return c

    lax.fori_loop(0, _RCAP // 16, _zr, 0)

    def _flush(cur):
        rdst[pl.ds(cur, 16)] = jnp.full((16,), _TRASH, jnp.int32)
        nseg = (cur + _SEG - 1) // _SEG

        def _gather(s, c):
            pltpu.async_copy(
                pf_hbm.at[rid.at[pl.ds(s * _SEG, _SEG)]],
                gbuf.at[pl.ds(s * _SEG, _SEG), :], sem)
            return c

        lax.fori_loop(0, nseg, _gather, 0)

        def _drain(s, c):
            pltpu.make_async_copy(
                pf_hbm.at[rid.at[pl.ds(0, _SEG)]],
                gbuf.at[pl.ds(0, _SEG), :], sem).wait()
            return c

        lax.fori_loop(0, nseg, _drain, 0)

        def _scat(g, c):
            base = rdst[pl.ds(g * 16, 16)]
            rows = iota + g * 16
            for dx in range(_PS):
                vals = plsc.load_gather(
                    gbuf, [rows, jnp.full((16,), dx, jnp.int32)])
                plsc.addupdate_scatter(canvas, [base + dx], vals)
            return c

        lax.fori_loop(0, (cur + 15) // 16, _scat, 0)
        return 0

    def _chunk(ci, cur_r):
        pltpu.async_copy(
            off_hbm.at[pl.ds(ci * _OCS, _OCS)], offs_v, sem).wait()

        def _vec(v, cur_k):
            ov = offs_v[pl.ds(v * 16, 16)]
            y0 = lax.shift_right_logical(ov, 9)
            m = (y0 >= w * _BH - (_PS - 1)) & (y0 <= w * _BH + (_BH - 1))
            plsc.store_compressed(klo.at[pl.ds(cur_k, 16)], ov, mask=m)
            plsc.store_compressed(
                klp.at[pl.ds(cur_k, 16)], iota + ci * _OCS + v * 16, mask=m)
            cnt = plsc.all_reduce_population_count(m)
            return cur_k + cnt[0]

        kcnt = lax.fori_loop(0, _OCS // 16, _vec, 0)
        klo[pl.ds(kcnt, 16)] = jnp.full((16,), 1 << 22, jnp.int32)
        kpad = (kcnt + 15) >> 4

        def _build(k, cur):
            cur = lax.cond(cur >= _RF, _flush, lambda c: c, cur)
            ov = klo[pl.ds(k * 16, 16)]
            pv = klp[pl.ds(k * 16, 16)]
            y0 = lax.shift_right_logical(ov, 9)
            x0 = ov & (_W - 1)
            pr = pv * (_C * _PS)
            for dy in range(_PS):
                y = y0 + dy
                m = lax.shift_right_logical(y, 4) == w
                d0 = ((y & (_BH - 1)) * _W) + x0
                cnt = plsc.all_reduce_population_count(m)[0]
                for ch in range(_C):
                    pos = cur + ch * cnt
                    plsc.store_compressed(
                        rid.at[pl.ds(pos, 16)], pr + (ch * _PS + dy), mask=m)
                    plsc.store_compressed(
                        rdst.at[pl.ds(pos, 16)], d0 + ch * (_BH * _W),
                        mask=m)
                cur = cur + _C * cnt
            return cur

        return lax.fori_loop(0, kpad, _build, cur_r)

    cur_r = lax.fori_loop(0, _NOC, _chunk, 0)

    @pl.when(cur_r > 0)
    def _():
        _flush(cur_r)

    pltpu.sync_copy(canvas.at[pl.ds(0, _BAND)], out_hbm.at[w])


_sc_fold = functools.partial(
    pl.kernel,
    out_type=jax.ShapeDtypeStruct((_NT, _BAND), jnp.float32),
    mesh=plsc.VectorSubcoreMesh(core_axis_name="c", subcore_axis_name="s"),
    compiler_params=pltpu.CompilerParams(
        needs_layout_passes=False, use_tc_tiling_on_sc=False),
    scratch_types=[
        pltpu.VMEM((_OCS,), jnp.int32),
        pltpu.VMEM((_OCS + 16,), jnp.int32),
        pltpu.VMEM((_OCS + 16,), jnp.int32),
        pltpu.VMEM((_RCAP,), jnp.int32),
        pltpu.VMEM((_RCAP + 16,), jnp.int32),
        pltpu.VMEM((_RCAP, _PS), jnp.float32),
        pltpu.VMEM((_BAND + 16,), jnp.float32),
        pltpu.SemaphoreType.DMA,
    ],
)(_sc_body)


def kernel(patches, inds):
    pf = patches.reshape(_NROWS, _PS)
    offs = _compute_offsets(inds.astype(jnp.int32))
    bands = _sc_fold(pf, offs)
    vid = bands.reshape(_NT, _C, _BH, _W).transpose(1, 0, 2, 3)
    return vid.reshape(1, _C, _H, _W)


# speedup vs baseline: 1.6208x; 1.6208x over previous
import functools

import jax
import jax.numpy as jnp
from jax import lax
from jax.experimental import pallas as pl
from jax.experimental.pallas import tpu as pltpu
from jax.experimental.pallas import tpu_sc as plsc

_PS = 16
_C = 3
_H = 512
_W = 512
_N = 65536
_WP = _W - _PS + 1
_NROWS = _N * _C * _PS

_NT = 32
_BH = _H // _NT
_BAND = _C * _BH * _W
_TRASH = _BAND

_OCS = 8192
_NOC = _N // _OCS
_RF = 2048
_RCAP = _RF + _C * _PS * 16
_SEG = 256


def _off_body(inds_ref, off_ref):
    i = inds_ref[...]
    y0 = i // _WP
    off_ref[...] = y0 * _W + (i - y0 * _WP)


def _compute_offsets(inds):
    inds2 = inds.reshape(_N // 128, 128)
    out = pl.pallas_call(
        _off_body,
        out_shape=jax.ShapeDtypeStruct((_N // 128, 128), jnp.int32),
    )(inds2)
    return out.reshape(_N)


def _sc_body(pf_hbm, off_hbm, out_hbm,
             offs_v, klo, klp, rid, rdst, gbuf, canvas, sem):
    cid = lax.axis_index("c")
    sid = lax.axis_index("s")
    w = cid * (_NT // 2) + sid
    iota = lax.broadcasted_iota(jnp.int32, (16,), 0)
    zeros16 = jnp.zeros((16,), jnp.float32)

    def _z(i, c):
        canvas[pl.ds(i * 16, 16)] = zeros16
        return c

    lax.fori_loop(0, (_BAND + 16) // 16, _z, 0)

    def _zr(i, c):
        rid[pl.ds(i * 16, 16)] = iota * 0
        return c

    lax.fori_loop(0, _RCAP // 16, _zr, 0)

    def _flush(cur):
        rdst[pl.ds(cur, 16)] = jnp.full((16,), _TRASH, jnp.int32)
        nseg = (cur + _SEG - 1) // _SEG

        def _gather(s, c):
            pltpu.async_copy(
                pf_hbm.at[rid.at[pl.ds(s * _SEG, _SEG)]],
                gbuf.at[pl.ds(s * _SEG, _SEG), :], sem)
            return c

        lax.fori_loop(0, nseg, _gather, 0)

        def _drain(s, c):
            pltpu.make_async_copy(
                pf_hbm.at[rid.at[pl.ds(0, _SEG)]],
                gbuf.at[pl.ds(0, _SEG), :], sem).wait()
            return c

        lax.fori_loop(0, nseg, _drain, 0)

        def _scat(g, c):
            base = rdst[pl.ds(g * 16, 16)]
            rows = iota + g * 16
            for dx in range(_PS):
                rot = (iota + dx) & (_PS - 1)
                vals = plsc.load_gather(gbuf, [rows, rot])
                plsc.addupdate_scatter(canvas, [base + rot], vals)
            return c

        lax.fori_loop(0, (cur + 15) // 16, _scat, 0)
        return 0

    def _chunk(ci, cur_r):
        pltpu.async_copy(
            off_hbm.at[pl.ds(ci * _OCS, _OCS)], offs_v, sem).wait()

        def _vec(v, cur_k):
            ov = offs_v[pl.ds(v * 16, 16)]
            y0 = lax.shift_right_logical(ov, 9)
            m = (y0 >= w * _BH - (_PS - 1)) & (y0 <= w * _BH + (_BH - 1))
            plsc.store_compressed(klo.at[pl.ds(cur_k, 16)], ov, mask=m)
            plsc.store_compressed(
                klp.at[pl.ds(cur_k, 16)], iota + ci * _OCS + v * 16, mask=m)
            cnt = plsc.all_reduce_population_count(m)
            return cur_k + cnt[0]

        kcnt = lax.fori_loop(0, _OCS // 16, _vec, 0)
        klo[pl.ds(kcnt, 16)] = jnp.full((16,), 1 << 22, jnp.int32)
        kpad = (kcnt + 15) >> 4

        def _build(k, cur):
            cur = lax.cond(cur >= _RF, _flush, lambda c: c, cur)
            ov = klo[pl.ds(k * 16, 16)]
            pv = klp[pl.ds(k * 16, 16)]
            y0 = lax.shift_right_logical(ov, 9)
            x0 = ov & (_W - 1)
            pr = pv * (_C * _PS)
            for dy in range(_PS):
                y = y0 + dy
                m = lax.shift_right_logical(y, 4) == w
                d0 = ((y & (_BH - 1)) * _W) + x0
                cnt = plsc.all_reduce_population_count(m)[0]
                for ch in range(_C):
                    pos = cur + ch * cnt
                    plsc.store_compressed(
                        rid.at[pl.ds(pos, 16)], pr + (ch * _PS + dy), mask=m)
                    plsc.store_compressed(
                        rdst.at[pl.ds(pos, 16)], d0 + ch * (_BH * _W),
                        mask=m)
                cur = cur + _C * cnt
            return cur

        return lax.fori_loop(0, kpad, _build, cur_r)

    cur_r = lax.fori_loop(0, _NOC, _chunk, 0)

    @pl.when(cur_r > 0)
    def _():
        _flush(cur_r)

    pltpu.sync_copy(canvas.at[pl.ds(0, _BAND)], out_hbm.at[w])


_sc_fold = functools.partial(
    pl.kernel,
    out_type=jax.ShapeDtypeStruct((_NT, _BAND), jnp.float32),
    mesh=plsc.VectorSubcoreMesh(core_axis_name="c", subcore_axis_name="s"),
    compiler_params=pltpu.CompilerParams(
        needs_layout_passes=False, use_tc_tiling_on_sc=False),
    scratch_types=[
        pltpu.VMEM((_OCS,), jnp.int32),
        pltpu.VMEM((_OCS + 16,), jnp.int32),
        pltpu.VMEM((_OCS + 16,), jnp.int32),
        pltpu.VMEM((_RCAP,), jnp.int32),
        pltpu.VMEM((_RCAP + 16,), jnp.int32),
        pltpu.VMEM((_RCAP, _PS), jnp.float32),
        pltpu.VMEM((_BAND + 16,), jnp.float32),
        pltpu.SemaphoreType.DMA,
    ],
)(_sc_body)


def kernel(patches, inds):
    pf = patches.reshape(_NROWS, _PS)
    offs = _compute_offsets(inds.astype(jnp.int32))
    bands = _sc_fold(pf, offs)
    vid = bands.reshape(_NT, _C, _BH, _W).transpose(1, 0, 2, 3)
    return vid.reshape(1, _C, _H, _W)
